# Initial kernel scaffold; baseline (speedup 1.0000x reference)
import jax
import jax.numpy as jnp
from jax.experimental import pallas as pl


def kernel(data, perturb, emb_W, emb_b, c1_W1, c1_b1, c1_g, c1_be, c1_W2, c1_b2, c2_W1, c2_b1, c2_g, c2_be, c2_W2, c2_b2, c3_W1, c3_b1, c3_g, c3_be, c3_W2, c3_b2, p1_w, p2_w, p3_w, ln1_g, ln1_b, ln2_g, ln2_b, ln3_g, ln3_b, ln4_g, ln4_b, l1_W, l1_b, l3_W, l3_b):
    raise NotImplementedError("write your pallas kernel here")


# speedup vs baseline: 2.8048x; 2.8048x over previous
import functools

import jax
import jax.numpy as jnp
from jax.experimental import pallas as pl
from jax.experimental.pallas import tpu as pltpu

N = 2048
DIN = 1024
H = 512
KNN_K = 50
NC = 128
RB = 256
NRB = N // RB
SB = 8
NSB = N // SB
F32 = jnp.float32
HIGHEST = jax.lax.Precision.HIGHEST


def _knn_kernel(xb_ref, xf_ref, m_ref, d_scr):
    i = pl.program_id(0)
    xb = xb_ref[...]
    xf = xf_ref[...]
    sqb = jnp.sum(xb * xb, axis=1, keepdims=True)
    sqf = jax.lax.dot_general(jnp.ones((1, DIN), F32), xf * xf,
                              (((1,), (1,)), ((), ())),
                              preferred_element_type=F32,
                              precision=HIGHEST)
    dot = jax.lax.dot_general(xb, xf, (((1,), (1,)), ((), ())),
                              preferred_element_type=F32)
    col = jax.lax.broadcasted_iota(jnp.int32, (RB, N), 1)
    row = jax.lax.broadcasted_iota(jnp.int32, (RB, N), 0) + i * RB
    d = sqb + sqf - 2.0 * dot
    d = jnp.where(col == row, -1.0, d)
    d_scr[...] = d
    m_ref[...] = jnp.zeros((RB, N), F32)

    def body(t, carry):
        dc = d_scr[...]
        cur = jnp.min(dc, axis=1, keepdims=True)
        eq = dc == cur
        idx = jnp.min(jnp.where(eq, col, N), axis=1, keepdims=True)
        sel = col == idx
        keep = jnp.where(t > 0, 1.0, 0.0)
        m_ref[...] += jnp.where(sel, keep, 0.0)
        d_scr[...] = jnp.where(sel, jnp.inf, dc)
        return carry

    jax.lax.fori_loop(0, KNN_K + 1, body, 0)


def _knn_mask(data):
    return pl.pallas_call(
        _knn_kernel,
        grid=(NRB,),
        in_specs=[pl.BlockSpec((RB, DIN), lambda i: (i, 0)),
                  pl.BlockSpec((N, DIN), lambda i: (0, 0))],
        out_specs=pl.BlockSpec((RB, N), lambda i: (i, 0)),
        out_shape=jax.ShapeDtypeStruct((N, N), F32),
        scratch_shapes=[pltpu.VMEM((RB, N), F32)],
    )(data, data)


def _embed_kernel(x_ref, w_ref, b_ref, o_ref):
    o_ref[...] = (jnp.dot(x_ref[...], w_ref[...], preferred_element_type=F32)
                  + b_ref[...])


def _embed(data, w, b):
    return pl.pallas_call(
        _embed_kernel,
        out_shape=jax.ShapeDtypeStruct((N, H), F32),
    )(data, w, b.reshape(1, H))


def _agg_kernel(m_ref, x_ref, v_ref, o_ref):
    i = pl.program_id(0)

    @pl.when(i == 0)
    def _init():
        o_ref[...] = jnp.zeros((N, H), F32)

    mt = jnp.transpose(m_ref[...])
    xv = x_ref[...] * v_ref[...]
    for j in range(SB):
        o_ref[...] += mt[:, j:j + 1] * xv[j:j + 1, :]


def _agg(m, x, v):
    return pl.pallas_call(
        _agg_kernel,
        grid=(NSB,),
        in_specs=[pl.BlockSpec((SB, N), lambda i: (i, 0)),
                  pl.BlockSpec((SB, H), lambda i: (i, 0)),
                  pl.BlockSpec((SB, 1), lambda i: (i, 0))],
        out_specs=pl.BlockSpec((N, H), lambda i: (0, 0)),
        out_shape=jax.ShapeDtypeStruct((N, H), F32),
    )(m, x, v)


def _leaky(t):
    return jnp.where(t >= 0.0, t, 0.1 * t)


def _gin_kernel(has_perturb, k_in, k_new, *refs):
    if has_perturb:
        (x_ref, a_ref, v_ref, pert_ref, w1_ref, b1_ref, g_ref, be_ref,
         w2_ref, b2_ref, lng_ref, lnb_ref, pw_ref,
         xo_ref, vo_ref, ro_ref) = refs
    else:
        (x_ref, a_ref, v_ref, w1_ref, b1_ref, g_ref, be_ref,
         w2_ref, b2_ref, lng_ref, lnb_ref, pw_ref,
         xo_ref, vo_ref, ro_ref) = refs
        pert_ref = None
    v = v_ref[...]
    h = (jnp.dot(x_ref[...] + a_ref[...], w1_ref[...],
                 preferred_element_type=F32) + b1_ref[...])
    inv_k = 1.0 / k_in
    mu = jnp.sum(h * v, axis=0, keepdims=True) * inv_k
    d0 = (h - mu) * v
    var = jnp.sum(d0 * d0, axis=0, keepdims=True) * inv_k
    hbn = (h - mu) / jnp.sqrt(var + 1e-5) * g_ref[...] + be_ref[...]
    hbn = jnp.maximum(hbn, 0.0)
    h2 = (jnp.dot(hbn, w2_ref[...], preferred_element_type=F32)
          + b2_ref[...])
    mu2 = jnp.mean(h2, axis=1, keepdims=True)
    dc = h2 - mu2
    var2 = jnp.mean(dc * dc, axis=1, keepdims=True)
    y = dc / jnp.sqrt(var2 + 1e-5) * lng_ref[...] + lnb_ref[...]
    if pert_ref is not None:
        y = y + pert_ref[...]
    y = _leaky(y)
    pw = pw_ref[...]
    y16 = y.astype(jnp.bfloat16).astype(F32)
    pw16 = pw.astype(jnp.bfloat16).astype(F32)
    s = jnp.tanh(jnp.sum(y16 * pw16, axis=1, keepdims=True)
                 / jnp.sqrt(jnp.sum(pw * pw)))
    row_i = jax.lax.broadcasted_iota(jnp.int32, (N, RB), 0)
    rank = jnp.zeros((N, 1), F32)
    for b in range(NRB):
        base = b * RB
        eb = (jax.lax.broadcasted_iota(jnp.int32, (N, RB), 0)
              == jax.lax.broadcasted_iota(jnp.int32, (N, RB), 1) + base
              ).astype(F32)
        s_blk = jax.lax.dot_general(s, eb, (((0,), (0,)), ((), ())),
                                    preferred_element_type=F32,
                                    precision=HIGHEST)
        v_blk = jax.lax.dot_general(v, eb, (((0,), (0,)), ((), ())),
                                    preferred_element_type=F32,
                                    precision=HIGHEST)
        j_ids = jax.lax.broadcasted_iota(jnp.int32, (N, RB), 1) + base
        beats = (s_blk > s) | ((s_blk == s) & (j_ids < row_i))
        rank += jnp.sum(jnp.where(beats, v_blk, 0.0), axis=1,
                        keepdims=True)
    nv = v * (rank < float(k_new)).astype(F32)
    xo = y * s
    xo_ref[...] = xo
    vo_ref[...] = nv
    xmax = jnp.max(jnp.where(nv > 0.0, xo, -jnp.inf), axis=0,
                   keepdims=True)
    xmean = jnp.sum(xo * nv, axis=0, keepdims=True) * (1.0 / k_new)
    ro_ref[:, :H] = xmax
    ro_ref[:, H:] = xmean


def _gin(x, agg, v, pert, w1, b1, g, be, w2, b2, lng, lnb, pw, k_in, k_new):
    has_perturb = pert is not None
    kern = functools.partial(_gin_kernel, has_perturb, k_in, k_new)
    args = [x, agg, v]
    if has_perturb:
        args.append(pert)
    args += [w1, b1.reshape(1, H), g.reshape(1, H), be.reshape(1, H), w2,
             b2.reshape(1, H), lng.reshape(1, H), lnb.reshape(1, H),
             pw.reshape(1, H)]
    return pl.pallas_call(
        kern,
        out_shape=(jax.ShapeDtypeStruct((N, H), F32),
                   jax.ShapeDtypeStruct((N, 1), F32),
                   jax.ShapeDtypeStruct((1, 2 * H), F32)),
    )(*args)


def _head_kernel(x1_ref, x2_ref, x3_ref, g4_ref, b4_ref, w1_ref, b1_ref,
                 w3_ref, b3_ref, f_ref, p_ref, s_ref):
    x3 = x3_ref[...]
    mu = jnp.mean(x3, axis=1, keepdims=True)
    d = x3 - mu
    var = jnp.mean(d * d, axis=1, keepdims=True)
    x3n = d / jnp.sqrt(var + 1e-5) * g4_ref[...] + b4_ref[...]
    xg = _leaky(x1_ref[...]) + _leaky(x2_ref[...]) + _leaky(x3n)
    feat = (jnp.dot(xg, w1_ref[...], preferred_element_type=F32,
                    precision=HIGHEST) + b1_ref[...])
    logits = (jnp.dot(feat, w3_ref[...], preferred_element_type=F32,
                      precision=HIGHEST) + b3_ref[...])
    z = logits - jnp.max(logits, axis=1, keepdims=True)
    ez = jnp.exp(z)
    p = ez / jnp.sum(ez, axis=1, keepdims=True)
    lg = jnp.log(1.0 - p)
    r_io = jax.lax.broadcasted_iota(jnp.int32, (NC, NC), 0)
    c_io = jax.lax.broadcasted_iota(jnp.int32, (NC, NC), 1)
    tmask = (r_io <= c_io).astype(F32)
    cs = jnp.dot(lg, tmask, preferred_element_type=F32,
                 precision=HIGHEST)
    f_ref[...] = feat
    p_ref[...] = p
    s_ref[...] = jnp.exp(cs)


def _head(x1, x2, x3, g4, b4, w1, b1, w3, b3):
    return pl.pallas_call(
        _head_kernel,
        out_shape=(jax.ShapeDtypeStruct((1, H), F32),
                   jax.ShapeDtypeStruct((1, NC), F32),
                   jax.ShapeDtypeStruct((1, NC), F32)),
    )(x1, x2, x3, g4.reshape(1, 2 * H), b4.reshape(1, 2 * H),
      w1, b1.reshape(1, H), w3, b3.reshape(1, NC))


def kernel(data, perturb, emb_W, emb_b, c1_W1, c1_b1, c1_g, c1_be, c1_W2,
           c1_b2, c2_W1, c2_b1, c2_g, c2_be, c2_W2, c2_b2, c3_W1, c3_b1,
           c3_g, c3_be, c3_W2, c3_b2, p1_w, p2_w, p3_w, ln1_g, ln1_b,
           ln2_g, ln2_b, ln3_g, ln3_b, ln4_g, ln4_b, l1_W, l1_b, l3_W,
           l3_b):
    m = _knn_mask(data)
    x = _embed(data, emb_W, emb_b)
    v1 = jnp.ones((N, 1), F32)

    a = _agg(m, x, v1)
    x, v2, x1 = _gin(x, a, v1, perturb, c1_W1, c1_b1, c1_g, c1_be, c1_W2,
                     c1_b2, ln1_g, ln1_b, p1_w, k_in=N, k_new=N // 2)

    a = _agg(m, x, v2)
    x, v3, x2 = _gin(x, a, v2, None, c2_W1, c2_b1, c2_g, c2_be, c2_W2,
                     c2_b2, ln2_g, ln2_b, p2_w, k_in=N // 2, k_new=N // 4)

    a = _agg(m, x, v3)
    x, v4, x3 = _gin(x, a, v3, None, c3_W1, c3_b1, c3_g, c3_be, c3_W2,
                     c3_b2, ln3_g, ln3_b, p3_w, k_in=N // 4, k_new=N // 8)

    return _head(x1, x2, x3, ln4_g, ln4_b, l1_W, l1_b, l3_W, l3_b)
